# baseline (speedup 1.0000x reference)
import functools

import jax
import jax.numpy as jnp
from jax import lax
from jax.experimental import pallas as pl
from jax.experimental.pallas import tpu as pltpu
from jax.experimental.pallas import tpu_sc as plsc

B = 16384
D = 32
CP = 20
CN = 100
C = CP + CN
NC, NS, L = 2, 16, 16
NW = NC * NS
BPW = B // NW
CB = 8
NCH = BPW // CB
NPOS = CB * CP
NNEG = CB * CN
CAND = NPOS + NNEG
NGN = 7


def _log_sigmoid(x):
    t = jnp.exp(-jnp.abs(x))
    s = t / (t + 2.0)
    s2 = s * s
    p = 1.0 + s2 * (0.33333334 + s2 * (0.2 + s2 * 0.14285715))
    return jnp.minimum(x, 0.0) - 2.0 * s * p


def _body(in_lbl, pos_lbl, neg_lbl, in_emb, out_emb, out,
          idx_in0, idx_in1, idx_cand0, idx_cand1,
          rows_in0, rows_in1, rows_cand0, rows_cand1,
          out_buf, dots, idx_sem0, idx_sem1, gat_sem0, gat_sem1):
    wid = lax.axis_index("s") * NC + lax.axis_index("c")
    b0 = wid * BPW

    idx_in = (idx_in0, idx_in1)
    idx_cand = (idx_cand0, idx_cand1)
    rows_in = (rows_in0, rows_in1)
    rows_cand = (rows_cand0, rows_cand1)
    idx_sem = (idx_sem0, idx_sem1)
    gat_sem = (gat_sem0, gat_sem1)

    iota = lax.iota(jnp.int32, L)
    sgn1 = jnp.where(iota < CP - L, 1.0, -1.0)
    valid8 = iota < C - 7 * L
    lane_last = iota == L - 1
    cidx = tuple(jnp.full((L,), c, jnp.int32) for c in range(C))

    def issue_idx(ch, p):
        boff = b0 + ch * CB
        pltpu.make_async_copy(in_lbl.at[pl.ds(boff, CB)], idx_in[p], idx_sem[p]).start()
        pltpu.make_async_copy(pos_lbl.at[pl.ds(boff * CP, NPOS)],
                              idx_cand[p].at[pl.ds(0, NPOS)], idx_sem[p]).start()
        pltpu.make_async_copy(neg_lbl.at[pl.ds(boff * CN, NNEG)],
                              idx_cand[p].at[pl.ds(NPOS, NNEG)], idx_sem[p]).start()

    def wait_idx(ch, p):
        boff = b0 + ch * CB
        pltpu.make_async_copy(in_lbl.at[pl.ds(boff, CB)], idx_in[p], idx_sem[p]).wait()
        pltpu.make_async_copy(pos_lbl.at[pl.ds(boff * CP, NPOS)],
                              idx_cand[p].at[pl.ds(0, NPOS)], idx_sem[p]).wait()
        pltpu.make_async_copy(neg_lbl.at[pl.ds(boff * CN, NNEG)],
                              idx_cand[p].at[pl.ds(NPOS, NNEG)], idx_sem[p]).wait()

    def issue_gather(p):
        pltpu.make_async_copy(in_emb.at[idx_in[p]], rows_in[p], gat_sem[p]).start()
        pltpu.make_async_copy(out_emb.at[idx_cand[p]], rows_cand[p], gat_sem[p]).start()

    def wait_gather(p):
        pltpu.make_async_copy(in_emb.at[idx_in[p]], rows_in[p], gat_sem[p]).wait()
        pltpu.make_async_copy(out_emb.at[idx_cand[p]], rows_cand[p], gat_sem[p]).wait()

    def process(ch, p):
        rc = rows_cand[p]
        ri = rows_in[p]

        def per_b(bl, carry):
            iv0 = ri[bl, pl.ds(0, L)]
            iv1 = ri[bl, pl.ds(L, L)]
            pb = bl * CP
            nb = NPOS + bl * CN
            for c in range(C):
                row = pb + c if c < CP else nb + (c - CP)
                t = rc[row, pl.ds(0, L)] * iv0 + rc[row, pl.ds(L, L)] * iv1
                plsc.store_scatter(dots, [cidx[c]], plsc.cumsum(t),
                                   mask=lane_last)
            total = _log_sigmoid(dots[pl.ds(0, L)])
            total = total + _log_sigmoid(dots[pl.ds(L, L)] * sgn1)
            for g in range(2, 7):
                total = total + _log_sigmoid(-dots[pl.ds(g * L, L)])
            total = total + jnp.where(
                valid8, _log_sigmoid(-dots[pl.ds(7 * L, L)]), 0.0)
            cs = jnp.cumsum(-total)
            oi = jnp.broadcast_to(ch * CB + bl, (L,))
            plsc.store_scatter(out_buf, [oi], cs, mask=lane_last)
            return carry

        lax.fori_loop(0, CB, per_b, 0, unroll=2)

    issue_idx(0, 0)
    issue_idx(1, 1)
    wait_idx(0, 0)
    issue_gather(0)

    def chunk_step(ch, p):
        q = 1 - p
        wait_gather(p)

        @pl.when(ch + 2 < NCH)
        def _():
            issue_idx(ch + 2, p)

        @pl.when(ch + 1 < NCH)
        def _():
            wait_idx(ch + 1, q)
            issue_gather(q)

        process(ch, p)

    def outer(j, carry):
        chunk_step(2 * j, 0)
        chunk_step(2 * j + 1, 1)
        return carry

    lax.fori_loop(0, NCH // 2, outer, 0)
    pltpu.sync_copy(out_buf, out.at[pl.ds(b0, BPW)])


@functools.partial(jax.jit)
def kernel(input_labels, pos_labels, neg_labels, in_embed, out_embed):
    pos_flat = pos_labels.reshape(B * CP)
    neg_flat = neg_labels.reshape(B * CN)
    mesh = plsc.VectorSubcoreMesh(core_axis_name="c", subcore_axis_name="s")
    f = pl.kernel(
        _body,
        out_type=jax.ShapeDtypeStruct((B,), jnp.float32),
        mesh=mesh,
        scratch_types=[
            pltpu.VMEM((CB,), jnp.int32), pltpu.VMEM((CB,), jnp.int32),
            pltpu.VMEM((CAND,), jnp.int32), pltpu.VMEM((CAND,), jnp.int32),
            pltpu.VMEM((CB, D), jnp.float32), pltpu.VMEM((CB, D), jnp.float32),
            pltpu.VMEM((CAND, D), jnp.float32), pltpu.VMEM((CAND, D), jnp.float32),
            pltpu.VMEM((BPW,), jnp.float32),
            pltpu.VMEM((8 * L,), jnp.float32),
            pltpu.SemaphoreType.DMA, pltpu.SemaphoreType.DMA,
            pltpu.SemaphoreType.DMA, pltpu.SemaphoreType.DMA,
        ],
        compiler_params=pltpu.CompilerParams(
            use_tc_tiling_on_sc=False, needs_layout_passes=False),
    )
    return f(input_labels, pos_flat, neg_flat, in_embed, out_embed)


# speedup vs baseline: 1.2498x; 1.2498x over previous
import functools

import jax
import jax.numpy as jnp
from jax import lax
from jax.experimental import pallas as pl
from jax.experimental.pallas import tpu as pltpu
from jax.experimental.pallas import tpu_sc as plsc

B = 16384
D = 32
CP = 20
CN = 100
C = CP + CN
NC, NS, L = 2, 16, 16
NW = NC * NS
BPW = B // NW
CB = 8
NCH = BPW // CB
NPOS = CB * CP
NNEG = CB * CN
CAND = NPOS + NNEG
NGN = 7
VOCAB_MAX = 999999


def _log_sigmoid(x):
    t = jnp.exp(-jnp.abs(x))
    s = t / (t + 2.0)
    s2 = s * s
    p = 1.0 + s2 * (0.33333334 + s2 * (0.2 + s2 * 0.14285715))
    return jnp.minimum(x, 0.0) - 2.0 * s * p


def _body(in_lbl, pos_lbl, neg_lbl, in_emb, out_emb, out,
          idxf_in0, idxf_in1, idxf_cand0, idxf_cand1,
          idx_in0, idx_in1, idx_cand0, idx_cand1,
          rows_in0, rows_in1, rows_cand0, rows_cand1,
          out_buf, dots, idx_sem0, idx_sem1, gat_sem0, gat_sem1):
    wid = lax.axis_index("s") * NC + lax.axis_index("c")
    b0 = wid * BPW

    idxf_in = (idxf_in0, idxf_in1)
    idxf_cand = (idxf_cand0, idxf_cand1)
    idx_in = (idx_in0, idx_in1)
    idx_cand = (idx_cand0, idx_cand1)
    rows_in = (rows_in0, rows_in1)
    rows_cand = (rows_cand0, rows_cand1)
    idx_sem = (idx_sem0, idx_sem1)
    gat_sem = (gat_sem0, gat_sem1)

    iota = lax.iota(jnp.int32, L)
    sgn1 = jnp.where(iota < CP - L, 1.0, -1.0)
    valid8 = iota < C - 7 * L
    lane_last = iota == L - 1
    cidx = tuple(jnp.full((L,), c, jnp.int32) for c in range(C))

    def issue_idx(ch, p):
        boff = b0 + ch * CB
        pltpu.make_async_copy(in_lbl.at[pl.ds(boff, CB)],
                              idxf_in[p].at[pl.ds(0, CB)], idx_sem[p]).start()
        pltpu.make_async_copy(pos_lbl.at[pl.ds(boff * CP, NPOS)],
                              idxf_cand[p].at[pl.ds(0, NPOS)], idx_sem[p]).start()
        pltpu.make_async_copy(neg_lbl.at[pl.ds(boff * CN, NNEG)],
                              idxf_cand[p].at[pl.ds(NPOS, NNEG)], idx_sem[p]).start()

    def wait_idx(ch, p):
        boff = b0 + ch * CB
        pltpu.make_async_copy(in_lbl.at[pl.ds(boff, CB)],
                              idxf_in[p].at[pl.ds(0, CB)], idx_sem[p]).wait()
        pltpu.make_async_copy(pos_lbl.at[pl.ds(boff * CP, NPOS)],
                              idxf_cand[p].at[pl.ds(0, NPOS)], idx_sem[p]).wait()
        pltpu.make_async_copy(neg_lbl.at[pl.ds(boff * CN, NNEG)],
                              idxf_cand[p].at[pl.ds(NPOS, NNEG)], idx_sem[p]).wait()

    def convert_idx(p):
        v = plsc.bitcast(idxf_in[p][...], jnp.int32)
        idx_in[p][...] = jnp.minimum(jnp.maximum(v, 0), VOCAB_MAX)
        for k in range(CAND // L):
            sl = pl.ds(k * L, L)
            idx_cand[p][sl] = plsc.bitcast(idxf_cand[p][sl], jnp.int32)

    def issue_gather(p):
        pltpu.make_async_copy(in_emb.at[idx_in[p]], rows_in[p], gat_sem[p]).start()
        pltpu.make_async_copy(out_emb.at[idx_cand[p]], rows_cand[p], gat_sem[p]).start()

    def wait_gather(p):
        pltpu.make_async_copy(in_emb.at[idx_in[p]], rows_in[p], gat_sem[p]).wait()
        pltpu.make_async_copy(out_emb.at[idx_cand[p]], rows_cand[p], gat_sem[p]).wait()

    def process(ch, p):
        rc = rows_cand[p]
        ri = rows_in[p]

        def per_b(bl, carry):
            iv0 = ri[bl, pl.ds(0, L)]
            iv1 = ri[bl, pl.ds(L, L)]
            pb = bl * CP
            nb = NPOS + bl * CN
            for c in range(C):
                row = pb + c if c < CP else nb + (c - CP)
                t = rc[row, pl.ds(0, L)] * iv0 + rc[row, pl.ds(L, L)] * iv1
                plsc.store_scatter(dots, [cidx[c]], plsc.cumsum(t),
                                   mask=lane_last)
            total = _log_sigmoid(dots[pl.ds(0, L)])
            total = total + _log_sigmoid(dots[pl.ds(L, L)] * sgn1)
            for g in range(2, 7):
                total = total + _log_sigmoid(-dots[pl.ds(g * L, L)])
            total = total + jnp.where(
                valid8, _log_sigmoid(-dots[pl.ds(7 * L, L)]), 0.0)
            cs = jnp.cumsum(-total)
            oi = jnp.broadcast_to(ch * CB + bl, (L,))
            plsc.store_scatter(out_buf, [oi], cs, mask=lane_last)
            return carry

        lax.fori_loop(0, CB, per_b, 0)

    issue_idx(0, 0)
    issue_idx(1, 1)
    wait_idx(0, 0)
    convert_idx(0)
    issue_gather(0)

    def chunk_step(ch, p):
        q = 1 - p
        wait_gather(p)

        @pl.when(ch + 2 < NCH)
        def _():
            issue_idx(ch + 2, p)

        @pl.when(ch + 1 < NCH)
        def _():
            wait_idx(ch + 1, q)
            convert_idx(q)
            issue_gather(q)

        process(ch, p)

    def outer(j, carry):
        chunk_step(2 * j, 0)
        chunk_step(2 * j + 1, 1)
        return carry

    lax.fori_loop(0, NCH // 2, outer, 0)
    pltpu.sync_copy(out_buf, out.at[pl.ds(b0, BPW)])


@functools.partial(jax.jit)
def kernel(input_labels, pos_labels, neg_labels, in_embed, out_embed):
    in_f = lax.bitcast_convert_type(input_labels, jnp.float32)
    pos_flat = lax.bitcast_convert_type(pos_labels.reshape(B * CP), jnp.float32)
    neg_flat = lax.bitcast_convert_type(neg_labels.reshape(B * CN), jnp.float32)
    mesh = plsc.VectorSubcoreMesh(core_axis_name="c", subcore_axis_name="s")
    f = pl.kernel(
        _body,
        out_type=jax.ShapeDtypeStruct((B,), jnp.float32),
        mesh=mesh,
        scratch_types=[
            pltpu.VMEM((L,), jnp.float32), pltpu.VMEM((L,), jnp.float32),
            pltpu.VMEM((CAND,), jnp.float32), pltpu.VMEM((CAND,), jnp.float32),
            pltpu.VMEM((L,), jnp.int32), pltpu.VMEM((L,), jnp.int32),
            pltpu.VMEM((CAND,), jnp.int32), pltpu.VMEM((CAND,), jnp.int32),
            pltpu.VMEM((L, D), jnp.float32), pltpu.VMEM((L, D), jnp.float32),
            pltpu.VMEM((CAND, D), jnp.float32), pltpu.VMEM((CAND, D), jnp.float32),
            pltpu.VMEM((BPW,), jnp.float32),
            pltpu.VMEM((8 * L,), jnp.float32),
            pltpu.SemaphoreType.DMA, pltpu.SemaphoreType.DMA,
            pltpu.SemaphoreType.DMA, pltpu.SemaphoreType.DMA,
        ],
        compiler_params=pltpu.CompilerParams(
            use_tc_tiling_on_sc=False, needs_layout_passes=False),
    )
    return f(in_f, pos_flat, neg_flat, in_embed, out_embed)


# speedup vs baseline: 1.9927x; 1.5943x over previous
import functools

import jax
import jax.numpy as jnp
from jax import lax
from jax.experimental import pallas as pl
from jax.experimental.pallas import tpu as pltpu
from jax.experimental.pallas import tpu_sc as plsc

B = 16384
D = 32
CP = 20
CN = 100
C = CP + CN
NC, NS, L = 2, 16, 16
NW = NC * NS
BPW = B // NW
CB = 8
NCH = BPW // CB
NPOS = CB * CP
NNEG = CB * CN
CAND = NPOS + NNEG
NGN = 7
VOCAB_MAX = 999999


def _log_sigmoid(x):
    t = jnp.exp(-jnp.abs(x))
    s = t / (t + 2.0)
    s2 = s * s
    p = 1.0 + s2 * (0.33333334 + s2 * (0.2 + s2 * 0.14285715))
    return jnp.minimum(x, 0.0) - 2.0 * s * p


def _body(in_lbl, pos_lbl, neg_lbl, in_emb, out_emb, out,
          idxf_in0, idxf_in1, idxf_cand0, idxf_cand1,
          idx_in0, idx_in1, idx_cand0, idx_cand1,
          rows_in0, rows_in1, rows_cand0, rows_cand1,
          out_buf, dots, idx_sem0, idx_sem1, gat_sem0, gat_sem1):
    wid = lax.axis_index("s") * NC + lax.axis_index("c")
    b0 = wid * BPW

    idxf_in = (idxf_in0, idxf_in1)
    idxf_cand = (idxf_cand0, idxf_cand1)
    idx_in = (idx_in0, idx_in1)
    idx_cand = (idx_cand0, idx_cand1)
    rows_in = (rows_in0, rows_in1)
    rows_cand = (rows_cand0, rows_cand1)
    idx_sem = (idx_sem0, idx_sem1)
    gat_sem = (gat_sem0, gat_sem1)

    iota = lax.iota(jnp.int32, L)
    sgn1 = jnp.where(iota < CP - L, 1.0, -1.0)
    valid8 = iota < C - 7 * L
    lane_last = iota == L - 1
    cidx = tuple(jnp.full((L,), c, jnp.int32) for c in range(C))

    def issue_idx(ch, p):
        boff = b0 + ch * CB
        pltpu.make_async_copy(in_lbl.at[pl.ds(boff, CB)],
                              idxf_in[p].at[pl.ds(0, CB)], idx_sem[p]).start()
        pltpu.make_async_copy(pos_lbl.at[pl.ds(boff * CP, NPOS)],
                              idxf_cand[p].at[pl.ds(0, NPOS)], idx_sem[p]).start()
        pltpu.make_async_copy(neg_lbl.at[pl.ds(boff * CN, NNEG)],
                              idxf_cand[p].at[pl.ds(NPOS, NNEG)], idx_sem[p]).start()

    def wait_idx(ch, p):
        boff = b0 + ch * CB
        pltpu.make_async_copy(in_lbl.at[pl.ds(boff, CB)],
                              idxf_in[p].at[pl.ds(0, CB)], idx_sem[p]).wait()
        pltpu.make_async_copy(pos_lbl.at[pl.ds(boff * CP, NPOS)],
                              idxf_cand[p].at[pl.ds(0, NPOS)], idx_sem[p]).wait()
        pltpu.make_async_copy(neg_lbl.at[pl.ds(boff * CN, NNEG)],
                              idxf_cand[p].at[pl.ds(NPOS, NNEG)], idx_sem[p]).wait()

    def convert_idx(p):
        v = plsc.bitcast(idxf_in[p][...], jnp.int32)
        idx_in[p][...] = jnp.minimum(jnp.maximum(v, 0), VOCAB_MAX)
        for k in range(CAND // L):
            sl = pl.ds(k * L, L)
            idx_cand[p][sl] = plsc.bitcast(idxf_cand[p][sl], jnp.int32)

    def issue_gather(p):
        pltpu.make_async_copy(in_emb.at[idx_in[p]], rows_in[p], gat_sem[p]).start()
        pltpu.make_async_copy(out_emb.at[idx_cand[p]], rows_cand[p], gat_sem[p]).start()

    def wait_gather(p):
        pltpu.make_async_copy(in_emb.at[idx_in[p]], rows_in[p], gat_sem[p]).wait()
        pltpu.make_async_copy(out_emb.at[idx_cand[p]], rows_cand[p], gat_sem[p]).wait()

    def process(ch, p):
        rc = rows_cand[p]
        ri = rows_in[p]

        def per_b(bl, carry):
            iv0 = ri[bl, pl.ds(0, L)]
            iv1 = ri[bl, pl.ds(L, L)]
            pb = bl * CP
            nb = NPOS + bl * CN - CP

            @plsc.parallel_loop(0, C, 1, unroll=8)
            def _dot(c):
                row = jnp.where(c < CP, pb + c, nb + c)
                t = rc[row, pl.ds(0, L)] * iv0 + rc[row, pl.ds(L, L)] * iv1
                plsc.store_scatter(dots, [jnp.broadcast_to(c, (L,))],
                                   plsc.cumsum(t), mask=lane_last)
            total = _log_sigmoid(dots[pl.ds(0, L)])
            total = total + _log_sigmoid(dots[pl.ds(L, L)] * sgn1)
            for g in range(2, 7):
                total = total + _log_sigmoid(-dots[pl.ds(g * L, L)])
            total = total + jnp.where(
                valid8, _log_sigmoid(-dots[pl.ds(7 * L, L)]), 0.0)
            cs = jnp.cumsum(-total)
            oi = jnp.broadcast_to(ch * CB + bl, (L,))
            plsc.store_scatter(out_buf, [oi], cs, mask=lane_last)
            return carry

        lax.fori_loop(0, CB, per_b, 0)

    issue_idx(0, 0)
    issue_idx(1, 1)
    wait_idx(0, 0)
    convert_idx(0)
    issue_gather(0)

    def chunk_step(ch, p):
        q = 1 - p
        wait_gather(p)

        @pl.when(ch + 2 < NCH)
        def _():
            issue_idx(ch + 2, p)

        @pl.when(ch + 1 < NCH)
        def _():
            wait_idx(ch + 1, q)
            convert_idx(q)
            issue_gather(q)

        process(ch, p)

    def outer(j, carry):
        chunk_step(2 * j, 0)
        chunk_step(2 * j + 1, 1)
        return carry

    lax.fori_loop(0, NCH // 2, outer, 0)
    pltpu.sync_copy(out_buf, out.at[pl.ds(b0, BPW)])


@functools.partial(jax.jit)
def kernel(input_labels, pos_labels, neg_labels, in_embed, out_embed):
    in_f = lax.bitcast_convert_type(input_labels, jnp.float32)
    pos_flat = lax.bitcast_convert_type(pos_labels.reshape(B * CP), jnp.float32)
    neg_flat = lax.bitcast_convert_type(neg_labels.reshape(B * CN), jnp.float32)
    mesh = plsc.VectorSubcoreMesh(core_axis_name="c", subcore_axis_name="s")
    f = pl.kernel(
        _body,
        out_type=jax.ShapeDtypeStruct((B,), jnp.float32),
        mesh=mesh,
        scratch_types=[
            pltpu.VMEM((L,), jnp.float32), pltpu.VMEM((L,), jnp.float32),
            pltpu.VMEM((CAND,), jnp.float32), pltpu.VMEM((CAND,), jnp.float32),
            pltpu.VMEM((L,), jnp.int32), pltpu.VMEM((L,), jnp.int32),
            pltpu.VMEM((CAND,), jnp.int32), pltpu.VMEM((CAND,), jnp.int32),
            pltpu.VMEM((L, D), jnp.float32), pltpu.VMEM((L, D), jnp.float32),
            pltpu.VMEM((CAND, D), jnp.float32), pltpu.VMEM((CAND, D), jnp.float32),
            pltpu.VMEM((BPW,), jnp.float32),
            pltpu.VMEM((8 * L,), jnp.float32),
            pltpu.SemaphoreType.DMA, pltpu.SemaphoreType.DMA,
            pltpu.SemaphoreType.DMA, pltpu.SemaphoreType.DMA,
        ],
        compiler_params=pltpu.CompilerParams(
            use_tc_tiling_on_sc=False, needs_layout_passes=False),
    )
    return f(in_f, pos_flat, neg_flat, in_embed, out_embed)
